# baseline (speedup 1.0000x reference)
import jax
import jax.numpy as jnp
from jax.experimental import pallas as pl
from jax.experimental.pallas import tpu as pltpu

_N_EMB = 32
_ROWS = 3072
_COLS = 50176
_RB = 128
_CB = 12544
_GRID = (_ROWS // _RB, _COLS // _CB)


def _body(x_ref, tbl_ref, o_ref):
    j = pl.program_id(1)
    part = jnp.sum(x_ref[...], axis=1, keepdims=True)

    @pl.when(j == 0)
    def _init():
        o_ref[...] = part

    @pl.when(j > 0)
    def _acc():
        o_ref[...] = o_ref[...] + part

    @pl.when(j == pl.num_programs(1) - 1)
    def _finalize():
        mean = o_ref[...] / float(_COLS)
        t = jnp.tanh(mean)
        idx = ((t + 1.0) / 2.0 * (_N_EMB - 1)).astype(jnp.int32)
        e = jax.lax.broadcasted_iota(jnp.int32, (1, _N_EMB), 1)
        onehot = (idx == e).astype(jnp.float32)
        o_ref[...] = jnp.sum(onehot * tbl_ref[...], axis=1, keepdims=True)


def kernel(x, beta_table):
    b, c = x.shape[0], x.shape[1]
    x2 = x.reshape(_ROWS, _COLS)
    tbl = beta_table.reshape(1, _N_EMB)
    out = pl.pallas_call(
        _body,
        grid=_GRID,
        in_specs=[
            pl.BlockSpec((_RB, _CB), lambda i, j: (i, j)),
            pl.BlockSpec((1, _N_EMB), lambda i, j: (0, 0)),
        ],
        out_specs=pl.BlockSpec((_RB, 1), lambda i, j: (i, 0)),
        out_shape=jax.ShapeDtypeStruct((_ROWS, 1), jnp.float32),
        compiler_params=pltpu.CompilerParams(
            dimension_semantics=("parallel", "arbitrary"),
        ),
    )(x2, tbl)
    return out.reshape(b, c, 1, 1)


# speedup vs baseline: 1.0018x; 1.0018x over previous
import jax
import jax.numpy as jnp
from jax.experimental import pallas as pl
from jax.experimental.pallas import tpu as pltpu

_N_EMB = 32
_ROWS = 3072
_COLS = 50176
_RB = 128
_CB = 12544
_GRID = (_ROWS // _RB, _COLS // _CB)
_LANES = 128
_CHUNKS = _CB // _LANES


def _body(x_ref, tbl_ref, o_ref, acc_ref):
    j = pl.program_id(1)

    x = x_ref[...]
    part = x[:, 0:_LANES]
    for k in range(1, _CHUNKS):
        part = part + x[:, k * _LANES:(k + 1) * _LANES]

    @pl.when(j == 0)
    def _init():
        acc_ref[...] = part

    @pl.when(j > 0)
    def _acc():
        acc_ref[...] = acc_ref[...] + part

    @pl.when(j == pl.num_programs(1) - 1)
    def _finalize():
        sums = jnp.sum(acc_ref[...], axis=1, keepdims=True)
        mean = sums / float(_COLS)
        t = jnp.tanh(mean)
        idx = ((t + 1.0) / 2.0 * (_N_EMB - 1)).astype(jnp.int32)
        e = jax.lax.broadcasted_iota(jnp.int32, (1, _N_EMB), 1)
        onehot = (idx == e).astype(jnp.float32)
        o_ref[...] = jnp.sum(onehot * tbl_ref[...], axis=1, keepdims=True)


def kernel(x, beta_table):
    b, c = x.shape[0], x.shape[1]
    x2 = x.reshape(_ROWS, _COLS)
    tbl = beta_table.reshape(1, _N_EMB)
    out = pl.pallas_call(
        _body,
        grid=_GRID,
        in_specs=[
            pl.BlockSpec((_RB, _CB), lambda i, j: (i, j)),
            pl.BlockSpec((1, _N_EMB), lambda i, j: (0, 0)),
        ],
        out_specs=pl.BlockSpec((_RB, 1), lambda i, j: (i, 0)),
        out_shape=jax.ShapeDtypeStruct((_ROWS, 1), jnp.float32),
        scratch_shapes=[pltpu.VMEM((_RB, _LANES), jnp.float32)],
        compiler_params=pltpu.CompilerParams(
            dimension_semantics=("parallel", "arbitrary"),
        ),
    )(x2, tbl)
    return out.reshape(b, c, 1, 1)


# speedup vs baseline: 1.9047x; 1.9014x over previous
import jax
import jax.numpy as jnp
from jax.experimental import pallas as pl
from jax.experimental.pallas import tpu as pltpu

_N_EMB = 32
_ROWS = 3072
_H = 224
_W = 224
_BC = 32
_GRID = (_ROWS // _BC,)


def _body(x_ref, tbl_ref, o_ref):
    sums = jnp.sum(x_ref[...], axis=(1, 2))
    mean = sums[:, None] / float(_H * _W)
    t = jnp.tanh(mean)
    idx = ((t + 1.0) / 2.0 * (_N_EMB - 1)).astype(jnp.int32)
    e = jax.lax.broadcasted_iota(jnp.int32, (1, _N_EMB), 1)
    onehot = (idx == e).astype(jnp.float32)
    o_ref[...] = jnp.sum(onehot * tbl_ref[...], axis=1, keepdims=True)


def kernel(x, beta_table):
    b, c = x.shape[0], x.shape[1]
    x3 = x.reshape(_ROWS, _H, _W)
    tbl = beta_table.reshape(1, _N_EMB)
    out = pl.pallas_call(
        _body,
        grid=_GRID,
        in_specs=[
            pl.BlockSpec((_BC, _H, _W), lambda i: (i, 0, 0)),
            pl.BlockSpec((1, _N_EMB), lambda i: (0, 0)),
        ],
        out_specs=pl.BlockSpec((_BC, 1), lambda i: (i, 0)),
        out_shape=jax.ShapeDtypeStruct((_ROWS, 1), jnp.float32),
        compiler_params=pltpu.CompilerParams(
            dimension_semantics=("parallel",),
        ),
    )(x3, tbl)
    return out.reshape(b, c, 1, 1)
